# baseline (speedup 1.0000x reference)
import jax
import jax.numpy as jnp
from jax import lax
from jax.experimental import pallas as pl
from jax.experimental.pallas import tpu as pltpu
from jax.experimental.pallas import tpu_sc as plsc

N = 50000
E = 800000
D = 64
DH = 32
G = 256
VOCAB = 3100
MTU = 1500
NB_CLASSES = 53
ITERS = 3

CHUNK = 128
N_CHUNKS = (N + CHUNK - 1) // CHUNK
N_TAIL = N - (N_CHUNKS - 1) * CHUNK
NSUB = 16

BLK = 8
E_CHUNKS = 6400
E_PAD = E_CHUNKS * CHUNK
CPT = E_CHUNKS // NSUB
BPT = CPT // BLK
AGG_ROWS = 50016
ROWS_PER_TILE = AGG_ROWS // NSUB
OUT_ROWS_PER_TILE = N // NSUB

BN = 1000
NB = N // BN

_mesh = plsc.VectorSubcoreMesh(core_axis_name="c", subcore_axis_name="s")
_sc_params = pltpu.CompilerParams(use_tc_tiling_on_sc=False)
_sc_params_nolayout = pltpu.CompilerParams(use_tc_tiling_on_sc=False,
                                           needs_layout_passes=False)


def _embed_body(emb2_hbm, idx2d_hbm, h2_out, idx_v, rows_v, sem):
    c = lax.axis_index("c")
    s = lax.axis_index("s")
    n_s = (N_CHUNKS - s + NSUB - 1) // NSUB

    def body(i, _):
        j = s + NSUB * i
        pltpu.sync_copy(idx2d_hbm.at[j], idx_v)
        pltpu.async_copy(emb2_hbm.at[c].at[idx_v], rows_v, sem).wait()

        @pl.when(j < N_CHUNKS - 1)
        def _():
            pltpu.sync_copy(rows_v, h2_out.at[c, pl.ds(j * CHUNK, CHUNK)])

        @pl.when(j == N_CHUNKS - 1)
        def _():
            pltpu.sync_copy(rows_v.at[pl.ds(0, N_TAIL)],
                            h2_out.at[c, pl.ds(j * CHUNK, N_TAIL)])
        return 0

    lax.fori_loop(0, n_s, body, 0)


_embed_call = pl.kernel(
    _embed_body,
    out_type=jax.ShapeDtypeStruct((2, N, DH), jnp.float32),
    mesh=_mesh,
    compiler_params=_sc_params,
    scratch_types=[
        pltpu.VMEM((CHUNK,), jnp.int32),
        pltpu.VMEM((CHUNK, DH), jnp.float32),
        pltpu.SemaphoreType.DMA,
    ],
)


def _make_edge_call(bf16_table, nbuf, gd):
    def pipeline(h2_hbm, src2d_hbm, dst2d_hbm, zeros_hbm, agg_out,
                 agg_sp, sidx2, didx2, rowsg, rowsf, isem, sems):
        c = lax.axis_index("c")
        s = lax.axis_index("s")
        gsem = sems[:nbuf]
        ssem = sems[nbuf:]
        base = s * CPT

        def gsrc(p, j):
            return h2_hbm.at[c].at[sidx2.at[p, j]]

        def sdst(p, k):
            return agg_sp.at[didx2.at[p, k]]

        def convert(ks):
            bfr = rowsg.at[ks]
            ffr = rowsf.at[ks]
            mask = jnp.full((16,), -65536, jnp.int32)

            def crow(r8, _):
                for q in range(8):
                    r = r8 * 8 + q
                    xi = plsc.bitcast(bfr[r], jnp.int32)
                    ffr[r, pl.ds(0, 16)] = plsc.bitcast(
                        lax.shift_left(xi, 16), jnp.float32)
                    ffr[r, pl.ds(16, 16)] = plsc.bitcast(
                        lax.bitwise_and(xi, mask), jnp.float32)
                return 0

            lax.fori_loop(0, CHUNK // 8, crow, 0)

        pltpu.sync_copy(zeros_hbm,
                        agg_sp.at[pl.ds(s * ROWS_PER_TILE, ROWS_PER_TILE)])
        plsc.subcore_barrier()

        pltpu.async_copy(src2d_hbm.at[pl.ds(base, BLK)], sidx2.at[0], isem)
        pltpu.async_copy(dst2d_hbm.at[pl.ds(base, BLK)], didx2.at[0], isem)

        def block(b, _):
            p = lax.rem(b, 3)
            boff = base + b * BLK
            pltpu.make_async_copy(src2d_hbm.at[pl.ds(boff, BLK)],
                                  sidx2.at[p], isem).wait()
            pltpu.make_async_copy(dst2d_hbm.at[pl.ds(boff, BLK)],
                                  didx2.at[p], isem).wait()

            @pl.when(b + 1 < BPT)
            def _():
                pn = lax.rem(b + 1, 3)
                noff = boff + BLK
                pltpu.async_copy(src2d_hbm.at[pl.ds(noff, BLK)],
                                 sidx2.at[pn], isem)
                pltpu.async_copy(dst2d_hbm.at[pl.ds(noff, BLK)],
                                 didx2.at[pn], isem)

            def scatter(p2, k):
                ks = k % nbuf
                pltpu.make_async_copy(gsrc(p2, k), rowsg.at[ks],
                                      gsem[ks]).wait()
                if bf16_table:
                    convert(ks)
                pltpu.async_copy(rowsf.at[ks], sdst(p2, k),
                                 ssem[ks], add=True)

            for j in range(BLK):
                slot = j % nbuf
                if j >= nbuf:
                    pltpu.make_async_copy(rowsf.at[slot],
                                          sdst(p, j - nbuf),
                                          ssem[slot]).wait()
                else:
                    @pl.when(b > 0)
                    def _(slot=slot, j=j, p=p):
                        pltpu.make_async_copy(rowsf.at[slot], sdst(p, j),
                                              ssem[slot]).wait()
                pltpu.async_copy(gsrc(p, j), rowsg.at[slot], gsem[slot])
                if j >= gd:
                    scatter(p, j - gd)
            for k in range(BLK - gd, BLK):
                scatter(p, k)
            return 0

        lax.fori_loop(0, BPT, block, 0)
        lastp = (BPT - 1) % 3
        for k in range(BLK - nbuf, BLK):
            ks = k % nbuf
            pltpu.make_async_copy(rowsf.at[ks], sdst(lastp, k),
                                  ssem[ks]).wait()
        plsc.subcore_barrier()
        pltpu.sync_copy(
            agg_sp.at[pl.ds(s * OUT_ROWS_PER_TILE, OUT_ROWS_PER_TILE)],
            agg_out.at[c, pl.ds(s * OUT_ROWS_PER_TILE, OUT_ROWS_PER_TILE)])

    gdtype = jnp.bfloat16 if bf16_table else jnp.float32
    scratch = [
        pltpu.VMEM_SHARED((AGG_ROWS, DH), jnp.float32),
        pltpu.VMEM((3, BLK, CHUNK), jnp.int32),
        pltpu.VMEM((3, BLK, CHUNK), jnp.int32),
        pltpu.VMEM((nbuf, CHUNK, DH), gdtype),
    ]
    if bf16_table:
        scratch.append(pltpu.VMEM((nbuf, CHUNK, DH), jnp.float32))
    scratch += [pltpu.SemaphoreType.DMA] * (1 + 2 * nbuf)

    if bf16_table:
        def body(h2_hbm, src2d_hbm, dst2d_hbm, zeros_hbm, agg_out,
                 agg_sp, sidx2, didx2, rowsg, rowsf, isem, *sems):
            pipeline(h2_hbm, src2d_hbm, dst2d_hbm, zeros_hbm, agg_out,
                     agg_sp, sidx2, didx2, rowsg, rowsf, isem, sems)
    else:
        def body(h2_hbm, src2d_hbm, dst2d_hbm, zeros_hbm, agg_out,
                 agg_sp, sidx2, didx2, rowsg, isem, *sems):
            pipeline(h2_hbm, src2d_hbm, dst2d_hbm, zeros_hbm, agg_out,
                     agg_sp, sidx2, didx2, rowsg, rowsg, isem, sems)

    return pl.kernel(
        body,
        out_type=jax.ShapeDtypeStruct((2, N, DH), jnp.float32),
        mesh=_mesh,
        compiler_params=_sc_params_nolayout if bf16_table else _sc_params,
        scratch_types=scratch,
    )


_edge_call_f32 = _make_edge_call(False, 6, 3)
_edge_call_bf16 = _make_edge_call(True, 4, 2)


def _mlp_body(h2_ref, agg_ref, w0_ref, b0_ref, w1_ref, b1_ref, w2_ref, b2_ref,
              eps_ref, u_ref, st_ref):
    i = pl.program_id(0)
    h = jnp.concatenate([h2_ref[0], h2_ref[1]], axis=-1)
    agg = jnp.concatenate([agg_ref[0], agg_ref[1]], axis=-1)
    z = (1.0 + eps_ref[0, 0]) * h + agg
    dn = (((1,), (1,)), ((), ()))
    z = jnp.maximum(lax.dot_general(z, w0_ref[...], dn,
                                    preferred_element_type=jnp.float32)
                    + b0_ref[...], 0.0)
    z = jnp.maximum(lax.dot_general(z, w1_ref[...], dn,
                                    preferred_element_type=jnp.float32)
                    + b1_ref[...], 0.0)
    z = jnp.maximum(lax.dot_general(z, w2_ref[...], dn,
                                    preferred_element_type=jnp.float32)
                    + b2_ref[...], 0.0)
    u_ref[...] = z
    st = jnp.concatenate([jnp.sum(z, axis=0, keepdims=True),
                          jnp.sum(z * z, axis=0, keepdims=True)], axis=0)

    @pl.when(i == 0)
    def _():
        st_ref[...] = st

    @pl.when(i > 0)
    def _():
        st_ref[...] += st


def _mlp_call(h2, agg2, W0, b0, W1, b1, W2, b2, eps):
    full = lambda shape: pl.BlockSpec(shape, lambda i: (0,) * len(shape))
    return pl.pallas_call(
        _mlp_body,
        grid=(NB,),
        in_specs=[
            pl.BlockSpec((2, BN, DH), lambda i: (0, i, 0)),
            pl.BlockSpec((2, BN, DH), lambda i: (0, i, 0)),
            full((D, D)), full((1, D)),
            full((D, D)), full((1, D)),
            full((D, D)), full((1, D)),
            full((1, 1)),
        ],
        out_specs=[
            pl.BlockSpec((BN, D), lambda i: (i, 0)),
            pl.BlockSpec((2, D), lambda i: (0, 0)),
        ],
        out_shape=[
            jax.ShapeDtypeStruct((N, D), jnp.float32),
            jax.ShapeDtypeStruct((2, D), jnp.float32),
        ],
    )(h2, agg2, W0, b0.reshape(1, D), W1, b1.reshape(1, D),
      W2, b2.reshape(1, D), eps.reshape(1, 1))


def _bn_pool_body(u_ref, st_ref, gamma_ref, beta_ref, gid_ref,
                  h2_ref, h2b_ref, gf_ref):
    i = pl.program_id(0)
    inv_n = 1.0 / N
    mean = st_ref[0:1, :] * inv_n
    var = st_ref[1:2, :] * inv_n - mean * mean
    scale = lax.rsqrt(var + 1e-5) * gamma_ref[...]
    h = (u_ref[...] - mean) * scale + beta_ref[...]
    h2_ref[0] = h[:, :DH]
    h2_ref[1] = h[:, DH:]
    for half in range(2):
        x = h[:, half * DH:(half + 1) * DH].astype(jnp.bfloat16)
        y = x.reshape(BN, 2, 16).transpose(0, 2, 1).reshape(BN, DH)
        h2b_ref[half] = y
    oh = (gid_ref[...] == lax.broadcasted_iota(jnp.int32, (1, G), 1)
          ).astype(jnp.float32)
    part = lax.dot_general(oh, h, (((0,), (0,)), ((), ())),
                           preferred_element_type=jnp.float32)

    @pl.when(i == 0)
    def _():
        gf_ref[...] = part

    @pl.when(i > 0)
    def _():
        gf_ref[...] += part


def _bn_pool_call(u, st, gamma, beta, gid2d):
    return pl.pallas_call(
        _bn_pool_body,
        grid=(NB,),
        in_specs=[
            pl.BlockSpec((BN, D), lambda i: (i, 0)),
            pl.BlockSpec((2, D), lambda i: (0, 0)),
            pl.BlockSpec((1, D), lambda i: (0, 0)),
            pl.BlockSpec((1, D), lambda i: (0, 0)),
            pl.BlockSpec((BN, 1), lambda i: (i, 0)),
        ],
        out_specs=[
            pl.BlockSpec((2, BN, DH), lambda i: (0, i, 0)),
            pl.BlockSpec((2, BN, DH), lambda i: (0, i, 0)),
            pl.BlockSpec((G, D), lambda i: (0, 0)),
        ],
        out_shape=[
            jax.ShapeDtypeStruct((2, N, DH), jnp.float32),
            jax.ShapeDtypeStruct((2, N, DH), jnp.bfloat16),
            jax.ShapeDtypeStruct((G, D), jnp.float32),
        ],
    )(u, st, gamma.reshape(1, D), beta.reshape(1, D), gid2d)


def _final_body(g0_ref, g1_ref, g2_ref, w_ref, b_ref, out_ref):
    dn = (((1,), (1,)), ((), ()))
    acc = lax.dot_general(g0_ref[...], w_ref[:, 0:D], dn,
                          preferred_element_type=jnp.float32)
    acc += lax.dot_general(g1_ref[...], w_ref[:, D:2 * D], dn,
                           preferred_element_type=jnp.float32)
    acc += lax.dot_general(g2_ref[...], w_ref[:, 2 * D:3 * D], dn,
                           preferred_element_type=jnp.float32)
    out_ref[...] = acc + b_ref[...]


def _final_call(g0, g1, g2, lin_W, lin_b):
    return pl.pallas_call(
        _final_body,
        out_shape=jax.ShapeDtypeStruct((G, NB_CLASSES), jnp.float32),
    )(g0, g1, g2, lin_W, lin_b.reshape(1, NB_CLASSES))


@jax.jit
def kernel(pkt_length, edge_index, node_graph_id, emb_table, W0, b0, W1, b1,
           W2, b2, bn_gamma, bn_beta, eps_gin, lin_W, lin_b):
    idx = (pkt_length + MTU).astype(jnp.int32)
    idx_pad = jnp.zeros((N_CHUNKS * CHUNK,), jnp.int32).at[:N].set(idx)
    idx2d = idx_pad.reshape(N_CHUNKS, CHUNK)
    npad = E_PAD - E
    src_pad = jnp.zeros((npad,), jnp.int32)
    dst_pad = N + (jnp.arange(npad, dtype=jnp.int32) % (AGG_ROWS - N))
    src2d = jnp.concatenate([edge_index[0].astype(jnp.int32), src_pad]
                            ).reshape(E_CHUNKS, CHUNK)
    dst2d = jnp.concatenate([edge_index[1].astype(jnp.int32), dst_pad]
                            ).reshape(E_CHUNKS, CHUNK)
    emb2 = emb_table.reshape(VOCAB, 2, DH).transpose(1, 0, 2)
    gid2d = node_graph_id.astype(jnp.int32).reshape(N, 1)
    zeros = jnp.zeros((ROWS_PER_TILE, DH), jnp.float32)

    h2 = _embed_call(emb2, idx2d)
    h2b = None
    gfs = []
    for it in range(ITERS):
        if it < ITERS - 1:
            agg2 = _edge_call_f32(h2, src2d, dst2d, zeros)
        else:
            agg2 = _edge_call_bf16(h2b, src2d, dst2d, zeros)
        u, st = _mlp_call(h2, agg2, W0, b0, W1, b1, W2, b2, eps_gin)
        h2, h2b, gf = _bn_pool_call(u, st, bn_gamma, bn_beta, gid2d)
        gfs.append(gf)
    return _final_call(gfs[0], gfs[1], gfs[2], lin_W, lin_b)


# speedup vs baseline: 1.4142x; 1.4142x over previous
import jax
import jax.numpy as jnp
from jax import lax
from jax.experimental import pallas as pl
from jax.experimental.pallas import tpu as pltpu
from jax.experimental.pallas import tpu_sc as plsc

N = 50000
E = 800000
D = 64
DH = 32
G = 256
VOCAB = 3100
MTU = 1500
NB_CLASSES = 53
ITERS = 3

CHUNK = 128
N_CHUNKS = (N + CHUNK - 1) // CHUNK
N_TAIL = N - (N_CHUNKS - 1) * CHUNK
NSUB = 16

BLK = 8
E_CHUNKS = 6400
E_PAD = E_CHUNKS * CHUNK
CPT = E_CHUNKS // NSUB
BPT = CPT // BLK
AGG_ROWS = 50016
ROWS_PER_TILE = AGG_ROWS // NSUB
OUT_ROWS_PER_TILE = N // NSUB

BN = 1000
NB = N // BN

_mesh = plsc.VectorSubcoreMesh(core_axis_name="c", subcore_axis_name="s")
_sc_params = pltpu.CompilerParams(use_tc_tiling_on_sc=False)
_sc_params_nolayout = pltpu.CompilerParams(use_tc_tiling_on_sc=False,
                                           needs_layout_passes=False)


def _embed_body(emb2_hbm, idx2d_hbm, h2_out, idx_v, rows_v, sem):
    c = lax.axis_index("c")
    s = lax.axis_index("s")
    n_s = (N_CHUNKS - s + NSUB - 1) // NSUB

    def body(i, _):
        j = s + NSUB * i
        pltpu.sync_copy(idx2d_hbm.at[j], idx_v)
        pltpu.async_copy(emb2_hbm.at[c].at[idx_v], rows_v, sem).wait()

        @pl.when(j < N_CHUNKS - 1)
        def _():
            pltpu.sync_copy(rows_v, h2_out.at[c, pl.ds(j * CHUNK, CHUNK)])

        @pl.when(j == N_CHUNKS - 1)
        def _():
            pltpu.sync_copy(rows_v.at[pl.ds(0, N_TAIL)],
                            h2_out.at[c, pl.ds(j * CHUNK, N_TAIL)])
        return 0

    lax.fori_loop(0, n_s, body, 0)


_embed_call = pl.kernel(
    _embed_body,
    out_type=jax.ShapeDtypeStruct((2, N, DH), jnp.float32),
    mesh=_mesh,
    compiler_params=_sc_params,
    scratch_types=[
        pltpu.VMEM((CHUNK,), jnp.int32),
        pltpu.VMEM((CHUNK, DH), jnp.float32),
        pltpu.SemaphoreType.DMA,
    ],
)


def _make_edge_call(bf16_table, nbuf, gd):
    def pipeline(h2_hbm, src2d_hbm, dst2d_hbm, zeros_hbm, agg_out,
                 agg_sp, sidx2, didx2, rowsg, rowsf, isem, sems):
        c = lax.axis_index("c")
        s = lax.axis_index("s")
        gsem = sems[:nbuf]
        ssem = sems[nbuf:]
        base = s * CPT

        def gsrc(p, j):
            return h2_hbm.at[c].at[sidx2.at[p, j]]

        def sdst(p, k):
            return agg_sp.at[didx2.at[p, k]]

        def convert(ks):
            bfr = rowsg.at[ks]
            ffr = rowsf.at[ks]
            mask = jnp.full((16,), -65536, jnp.int32)

            def crow(r8, _):
                for q in range(8):
                    r = r8 * 8 + q
                    xi = plsc.bitcast(bfr[r], jnp.int32)
                    ffr[r, pl.ds(0, 16)] = plsc.bitcast(
                        lax.shift_left(xi, 16), jnp.float32)
                    ffr[r, pl.ds(16, 16)] = plsc.bitcast(
                        lax.bitwise_and(xi, mask), jnp.float32)
                return 0

            lax.fori_loop(0, CHUNK // 8, crow, 0)

        pltpu.sync_copy(zeros_hbm,
                        agg_sp.at[pl.ds(s * ROWS_PER_TILE, ROWS_PER_TILE)])
        plsc.subcore_barrier()

        pltpu.async_copy(src2d_hbm.at[pl.ds(base, BLK)], sidx2.at[0], isem)
        pltpu.async_copy(dst2d_hbm.at[pl.ds(base, BLK)], didx2.at[0], isem)

        def block(b, _):
            p = lax.rem(b, 3)
            boff = base + b * BLK
            pltpu.make_async_copy(src2d_hbm.at[pl.ds(boff, BLK)],
                                  sidx2.at[p], isem).wait()
            pltpu.make_async_copy(dst2d_hbm.at[pl.ds(boff, BLK)],
                                  didx2.at[p], isem).wait()

            @pl.when(b + 1 < BPT)
            def _():
                pn = lax.rem(b + 1, 3)
                noff = boff + BLK
                pltpu.async_copy(src2d_hbm.at[pl.ds(noff, BLK)],
                                 sidx2.at[pn], isem)
                pltpu.async_copy(dst2d_hbm.at[pl.ds(noff, BLK)],
                                 didx2.at[pn], isem)

            def scatter(p2, k):
                ks = k % nbuf
                pltpu.make_async_copy(gsrc(p2, k), rowsg.at[ks],
                                      gsem[ks]).wait()
                if bf16_table:
                    convert(ks)
                pltpu.async_copy(rowsf.at[ks], sdst(p2, k),
                                 ssem[ks], add=True)

            for j in range(BLK):
                slot = j % nbuf
                if j >= nbuf:
                    pltpu.make_async_copy(rowsf.at[slot],
                                          sdst(p, j - nbuf),
                                          ssem[slot]).wait()
                else:
                    @pl.when(b > 0)
                    def _(slot=slot, j=j, p=p):
                        pltpu.make_async_copy(rowsf.at[slot], sdst(p, j),
                                              ssem[slot]).wait()
                pltpu.async_copy(gsrc(p, j), rowsg.at[slot], gsem[slot])
                if j >= gd:
                    scatter(p, j - gd)
            for k in range(BLK - gd, BLK):
                scatter(p, k)
            return 0

        lax.fori_loop(0, BPT, block, 0)
        lastp = (BPT - 1) % 3
        for k in range(BLK - nbuf, BLK):
            ks = k % nbuf
            pltpu.make_async_copy(rowsf.at[ks], sdst(lastp, k),
                                  ssem[ks]).wait()
        plsc.subcore_barrier()
        pltpu.sync_copy(
            agg_sp.at[pl.ds(s * OUT_ROWS_PER_TILE, OUT_ROWS_PER_TILE)],
            agg_out.at[c, pl.ds(s * OUT_ROWS_PER_TILE, OUT_ROWS_PER_TILE)])

    gdtype = jnp.bfloat16 if bf16_table else jnp.float32
    scratch = [
        pltpu.VMEM_SHARED((AGG_ROWS, DH), jnp.float32),
        pltpu.VMEM((3, BLK, CHUNK), jnp.int32),
        pltpu.VMEM((3, BLK, CHUNK), jnp.int32),
        pltpu.VMEM((nbuf, CHUNK, DH), gdtype),
    ]
    if bf16_table:
        scratch.append(pltpu.VMEM((nbuf, CHUNK, DH), jnp.float32))
    scratch += [pltpu.SemaphoreType.DMA] * (1 + 2 * nbuf)

    if bf16_table:
        def body(h2_hbm, src2d_hbm, dst2d_hbm, zeros_hbm, agg_out,
                 agg_sp, sidx2, didx2, rowsg, rowsf, isem, *sems):
            pipeline(h2_hbm, src2d_hbm, dst2d_hbm, zeros_hbm, agg_out,
                     agg_sp, sidx2, didx2, rowsg, rowsf, isem, sems)
    else:
        def body(h2_hbm, src2d_hbm, dst2d_hbm, zeros_hbm, agg_out,
                 agg_sp, sidx2, didx2, rowsg, isem, *sems):
            pipeline(h2_hbm, src2d_hbm, dst2d_hbm, zeros_hbm, agg_out,
                     agg_sp, sidx2, didx2, rowsg, rowsg, isem, sems)

    return pl.kernel(
        body,
        out_type=jax.ShapeDtypeStruct((2, N, DH), jnp.float32),
        mesh=_mesh,
        compiler_params=_sc_params_nolayout if bf16_table else _sc_params,
        scratch_types=scratch,
    )


_edge_call_f32 = _make_edge_call(False, 6, 3)
_edge_call_bf16 = _make_edge_call(True, 4, 2)


def _mlp_body(h2_ref, agg_ref, w0_ref, b0_ref, w1_ref, b1_ref, w2_ref, b2_ref,
              eps_ref, u_ref, st_ref):
    i = pl.program_id(0)
    h = jnp.concatenate([h2_ref[0], h2_ref[1]], axis=-1)
    agg = jnp.concatenate([agg_ref[0], agg_ref[1]], axis=-1)
    z = (1.0 + eps_ref[0, 0]) * h + agg
    dn = (((1,), (1,)), ((), ()))
    z = jnp.maximum(lax.dot_general(z, w0_ref[...], dn,
                                    preferred_element_type=jnp.float32)
                    + b0_ref[...], 0.0)
    z = jnp.maximum(lax.dot_general(z, w1_ref[...], dn,
                                    preferred_element_type=jnp.float32)
                    + b1_ref[...], 0.0)
    z = jnp.maximum(lax.dot_general(z, w2_ref[...], dn,
                                    preferred_element_type=jnp.float32)
                    + b2_ref[...], 0.0)
    u_ref[...] = z
    st = jnp.concatenate([jnp.sum(z, axis=0, keepdims=True),
                          jnp.sum(z * z, axis=0, keepdims=True)], axis=0)

    @pl.when(i == 0)
    def _():
        st_ref[...] = st

    @pl.when(i > 0)
    def _():
        st_ref[...] += st


def _mlp_call(h2, agg2, W0, b0, W1, b1, W2, b2, eps):
    full = lambda shape: pl.BlockSpec(shape, lambda i: (0,) * len(shape))
    return pl.pallas_call(
        _mlp_body,
        grid=(NB,),
        in_specs=[
            pl.BlockSpec((2, BN, DH), lambda i: (0, i, 0)),
            pl.BlockSpec((2, BN, DH), lambda i: (0, i, 0)),
            full((D, D)), full((1, D)),
            full((D, D)), full((1, D)),
            full((D, D)), full((1, D)),
            full((1, 1)),
        ],
        out_specs=[
            pl.BlockSpec((BN, D), lambda i: (i, 0)),
            pl.BlockSpec((2, D), lambda i: (0, 0)),
        ],
        out_shape=[
            jax.ShapeDtypeStruct((N, D), jnp.float32),
            jax.ShapeDtypeStruct((2, D), jnp.float32),
        ],
    )(h2, agg2, W0, b0.reshape(1, D), W1, b1.reshape(1, D),
      W2, b2.reshape(1, D), eps.reshape(1, 1))


def _bn_pool_body(u_ref, st_ref, gamma_ref, beta_ref, gid_ref, perm_ref,
                  h2_ref, h2b_ref, gf_ref):
    i = pl.program_id(0)
    inv_n = 1.0 / N
    mean = st_ref[0:1, :] * inv_n
    var = st_ref[1:2, :] * inv_n - mean * mean
    scale = lax.rsqrt(var + 1e-5) * gamma_ref[...]
    h = (u_ref[...] - mean) * scale + beta_ref[...]
    h2_ref[0] = h[:, :DH]
    h2_ref[1] = h[:, DH:]
    for half in range(2):
        x = h[:, half * DH:(half + 1) * DH]
        y = lax.dot_general(x, perm_ref[...], (((1,), (0,)), ((), ())),
                            preferred_element_type=jnp.float32)
        h2b_ref[half] = y.astype(jnp.bfloat16)
    oh = (gid_ref[...] == lax.broadcasted_iota(jnp.int32, (1, G), 1)
          ).astype(jnp.float32)
    part = lax.dot_general(oh, h, (((0,), (0,)), ((), ())),
                           preferred_element_type=jnp.float32)

    @pl.when(i == 0)
    def _():
        gf_ref[...] = part

    @pl.when(i > 0)
    def _():
        gf_ref[...] += part


def _bn_pool_call(u, st, gamma, beta, gid2d, perm):
    return pl.pallas_call(
        _bn_pool_body,
        grid=(NB,),
        in_specs=[
            pl.BlockSpec((BN, D), lambda i: (i, 0)),
            pl.BlockSpec((2, D), lambda i: (0, 0)),
            pl.BlockSpec((1, D), lambda i: (0, 0)),
            pl.BlockSpec((1, D), lambda i: (0, 0)),
            pl.BlockSpec((BN, 1), lambda i: (i, 0)),
            pl.BlockSpec((DH, DH), lambda i: (0, 0)),
        ],
        out_specs=[
            pl.BlockSpec((2, BN, DH), lambda i: (0, i, 0)),
            pl.BlockSpec((2, BN, DH), lambda i: (0, i, 0)),
            pl.BlockSpec((G, D), lambda i: (0, 0)),
        ],
        out_shape=[
            jax.ShapeDtypeStruct((2, N, DH), jnp.float32),
            jax.ShapeDtypeStruct((2, N, DH), jnp.bfloat16),
            jax.ShapeDtypeStruct((G, D), jnp.float32),
        ],
    )(u, st, gamma.reshape(1, D), beta.reshape(1, D), gid2d, perm)


def _final_body(g0_ref, g1_ref, g2_ref, w_ref, b_ref, out_ref):
    dn = (((1,), (1,)), ((), ()))
    acc = lax.dot_general(g0_ref[...], w_ref[:, 0:D], dn,
                          preferred_element_type=jnp.float32)
    acc += lax.dot_general(g1_ref[...], w_ref[:, D:2 * D], dn,
                           preferred_element_type=jnp.float32)
    acc += lax.dot_general(g2_ref[...], w_ref[:, 2 * D:3 * D], dn,
                           preferred_element_type=jnp.float32)
    out_ref[...] = acc + b_ref[...]


def _final_call(g0, g1, g2, lin_W, lin_b):
    return pl.pallas_call(
        _final_body,
        out_shape=jax.ShapeDtypeStruct((G, NB_CLASSES), jnp.float32),
    )(g0, g1, g2, lin_W, lin_b.reshape(1, NB_CLASSES))


@jax.jit
def kernel(pkt_length, edge_index, node_graph_id, emb_table, W0, b0, W1, b1,
           W2, b2, bn_gamma, bn_beta, eps_gin, lin_W, lin_b):
    idx = (pkt_length + MTU).astype(jnp.int32)
    idx_pad = jnp.zeros((N_CHUNKS * CHUNK,), jnp.int32).at[:N].set(idx)
    idx2d = idx_pad.reshape(N_CHUNKS, CHUNK)
    npad = E_PAD - E
    src_pad = jnp.zeros((npad,), jnp.int32)
    dst_pad = N + (jnp.arange(npad, dtype=jnp.int32) % (AGG_ROWS - N))
    src2d = jnp.concatenate([edge_index[0].astype(jnp.int32), src_pad]
                            ).reshape(E_CHUNKS, CHUNK)
    dst2d = jnp.concatenate([edge_index[1].astype(jnp.int32), dst_pad]
                            ).reshape(E_CHUNKS, CHUNK)
    emb2 = emb_table.reshape(VOCAB, 2, DH).transpose(1, 0, 2)
    gid2d = node_graph_id.astype(jnp.int32).reshape(N, 1)
    zeros = jnp.zeros((ROWS_PER_TILE, DH), jnp.float32)
    ar16 = jnp.arange(16, dtype=jnp.int32)
    psrc = jnp.stack([ar16, ar16 + 16], axis=1).reshape(DH)
    perm = jax.nn.one_hot(psrc, DH, axis=-1, dtype=jnp.float32).T

    h2 = _embed_call(emb2, idx2d)
    h2b = None
    gfs = []
    for it in range(ITERS):
        if it < ITERS - 1:
            agg2 = _edge_call_f32(h2, src2d, dst2d, zeros)
        else:
            agg2 = _edge_call_bf16(h2b, src2d, dst2d, zeros)
        u, st = _mlp_call(h2, agg2, W0, b0, W1, b1, W2, b2, eps_gin)
        h2, h2b, gf = _bn_pool_call(u, st, bn_gamma, bn_beta, gid2d, perm)
        gfs.append(gf)
    return _final_call(gfs[0], gfs[1], gfs[2], lin_W, lin_b)


# speedup vs baseline: 1.5981x; 1.1300x over previous
import jax
import jax.numpy as jnp
from jax import lax
from jax.experimental import pallas as pl
from jax.experimental.pallas import tpu as pltpu
from jax.experimental.pallas import tpu_sc as plsc

N = 50000
E = 800000
D = 64
DH = 32
G = 256
VOCAB = 3100
MTU = 1500
NB_CLASSES = 53
ITERS = 3

CHUNK = 128
N_CHUNKS = (N + CHUNK - 1) // CHUNK
N_TAIL = N - (N_CHUNKS - 1) * CHUNK
NSUB = 16

BLK = 8
E_CHUNKS = 6400
E_PAD = E_CHUNKS * CHUNK
CPT = E_CHUNKS // NSUB
BPT = CPT // BLK
AGG_ROWS = 50016
ROWS_PER_TILE = AGG_ROWS // NSUB
OUT_ROWS_PER_TILE = N // NSUB

BN = 2000
NB = N // BN

_mesh = plsc.VectorSubcoreMesh(core_axis_name="c", subcore_axis_name="s")
_sc_params = pltpu.CompilerParams(use_tc_tiling_on_sc=False)
_sc_params_nolayout = pltpu.CompilerParams(use_tc_tiling_on_sc=False,
                                           needs_layout_passes=False)


def _embed_body(emb2_hbm, idx2d_hbm, h2_out, idx_v, rows_v, sem):
    c = lax.axis_index("c")
    s = lax.axis_index("s")
    n_s = (N_CHUNKS - s + NSUB - 1) // NSUB

    def body(i, _):
        j = s + NSUB * i
        pltpu.sync_copy(idx2d_hbm.at[j], idx_v)
        pltpu.async_copy(emb2_hbm.at[c].at[idx_v], rows_v, sem).wait()

        @pl.when(j < N_CHUNKS - 1)
        def _():
            pltpu.sync_copy(rows_v, h2_out.at[c, pl.ds(j * CHUNK, CHUNK)])

        @pl.when(j == N_CHUNKS - 1)
        def _():
            pltpu.sync_copy(rows_v.at[pl.ds(0, N_TAIL)],
                            h2_out.at[c, pl.ds(j * CHUNK, N_TAIL)])
        return 0

    lax.fori_loop(0, n_s, body, 0)


_embed_call = pl.kernel(
    _embed_body,
    out_type=jax.ShapeDtypeStruct((2, N, DH), jnp.float32),
    mesh=_mesh,
    compiler_params=_sc_params,
    scratch_types=[
        pltpu.VMEM((CHUNK,), jnp.int32),
        pltpu.VMEM((CHUNK, DH), jnp.float32),
        pltpu.SemaphoreType.DMA,
    ],
)


def _make_edge_call(bf16_table, nbuf, gd):
    def pipeline(h2_hbm, src2d_hbm, dst2d_hbm, zeros_hbm, agg_out,
                 agg_sp, sidx2, didx2, rowsg, rowsf, isem, sems):
        c = lax.axis_index("c")
        s = lax.axis_index("s")
        gsem = sems[:nbuf]
        ssem = sems[nbuf:]
        base = s * CPT

        def gsrc(p, j):
            return h2_hbm.at[c].at[sidx2.at[p, j]]

        def sdst(p, k):
            return agg_sp.at[didx2.at[p, k]]

        def convert(ks):
            bfr = rowsg.at[ks]
            ffr = rowsf.at[ks]
            mask = jnp.full((16,), -65536, jnp.int32)

            def crow(r8, _):
                for q in range(8):
                    r = r8 * 8 + q
                    xi = plsc.bitcast(bfr[r], jnp.int32)
                    ffr[r, pl.ds(0, 16)] = plsc.bitcast(
                        lax.shift_left(xi, 16), jnp.float32)
                    ffr[r, pl.ds(16, 16)] = plsc.bitcast(
                        lax.bitwise_and(xi, mask), jnp.float32)
                return 0

            lax.fori_loop(0, CHUNK // 8, crow, 0)

        pltpu.sync_copy(zeros_hbm,
                        agg_sp.at[pl.ds(s * ROWS_PER_TILE, ROWS_PER_TILE)])
        plsc.subcore_barrier()

        pltpu.async_copy(src2d_hbm.at[pl.ds(base, BLK)], sidx2.at[0], isem)
        pltpu.async_copy(dst2d_hbm.at[pl.ds(base, BLK)], didx2.at[0], isem)

        def block(b, _):
            p = lax.rem(b, 3)
            boff = base + b * BLK
            pltpu.make_async_copy(src2d_hbm.at[pl.ds(boff, BLK)],
                                  sidx2.at[p], isem).wait()
            pltpu.make_async_copy(dst2d_hbm.at[pl.ds(boff, BLK)],
                                  didx2.at[p], isem).wait()

            @pl.when(b + 1 < BPT)
            def _():
                pn = lax.rem(b + 1, 3)
                noff = boff + BLK
                pltpu.async_copy(src2d_hbm.at[pl.ds(noff, BLK)],
                                 sidx2.at[pn], isem)
                pltpu.async_copy(dst2d_hbm.at[pl.ds(noff, BLK)],
                                 didx2.at[pn], isem)

            def scatter(p2, k):
                ks = k % nbuf
                pltpu.make_async_copy(gsrc(p2, k), rowsg.at[ks],
                                      gsem[ks]).wait()
                if bf16_table:
                    convert(ks)
                pltpu.async_copy(rowsf.at[ks], sdst(p2, k),
                                 ssem[ks], add=True)

            for j in range(BLK):
                slot = j % nbuf
                if j >= nbuf:
                    pltpu.make_async_copy(rowsf.at[slot],
                                          sdst(p, j - nbuf),
                                          ssem[slot]).wait()
                else:
                    @pl.when(b > 0)
                    def _(slot=slot, j=j, p=p):
                        pltpu.make_async_copy(rowsf.at[slot], sdst(p, j),
                                              ssem[slot]).wait()
                pltpu.async_copy(gsrc(p, j), rowsg.at[slot], gsem[slot])
                if j >= gd:
                    scatter(p, j - gd)
            for k in range(BLK - gd, BLK):
                scatter(p, k)
            return 0

        lax.fori_loop(0, BPT, block, 0)
        lastp = (BPT - 1) % 3
        for k in range(BLK - nbuf, BLK):
            ks = k % nbuf
            pltpu.make_async_copy(rowsf.at[ks], sdst(lastp, k),
                                  ssem[ks]).wait()
        plsc.subcore_barrier()
        pltpu.sync_copy(
            agg_sp.at[pl.ds(s * OUT_ROWS_PER_TILE, OUT_ROWS_PER_TILE)],
            agg_out.at[c, pl.ds(s * OUT_ROWS_PER_TILE, OUT_ROWS_PER_TILE)])

    gdtype = jnp.bfloat16 if bf16_table else jnp.float32
    scratch = [
        pltpu.VMEM_SHARED((AGG_ROWS, DH), jnp.float32),
        pltpu.VMEM((3, BLK, CHUNK), jnp.int32),
        pltpu.VMEM((3, BLK, CHUNK), jnp.int32),
        pltpu.VMEM((nbuf, CHUNK, DH), gdtype),
    ]
    if bf16_table:
        scratch.append(pltpu.VMEM((nbuf, CHUNK, DH), jnp.float32))
    scratch += [pltpu.SemaphoreType.DMA] * (1 + 2 * nbuf)

    if bf16_table:
        def body(h2_hbm, src2d_hbm, dst2d_hbm, zeros_hbm, agg_out,
                 agg_sp, sidx2, didx2, rowsg, rowsf, isem, *sems):
            pipeline(h2_hbm, src2d_hbm, dst2d_hbm, zeros_hbm, agg_out,
                     agg_sp, sidx2, didx2, rowsg, rowsf, isem, sems)
    else:
        def body(h2_hbm, src2d_hbm, dst2d_hbm, zeros_hbm, agg_out,
                 agg_sp, sidx2, didx2, rowsg, isem, *sems):
            pipeline(h2_hbm, src2d_hbm, dst2d_hbm, zeros_hbm, agg_out,
                     agg_sp, sidx2, didx2, rowsg, rowsg, isem, sems)

    return pl.kernel(
        body,
        out_type=jax.ShapeDtypeStruct((2, N, DH), jnp.float32),
        mesh=_mesh,
        compiler_params=_sc_params_nolayout if bf16_table else _sc_params,
        scratch_types=scratch,
    )


_edge_call_f32 = _make_edge_call(False, 6, 3)
_edge_call_bf16 = _make_edge_call(True, 4, 2)


def _mlp_body(h2_ref, agg_ref, w0_ref, b0_ref, w1_ref, b1_ref, w2_ref, b2_ref,
              eps_ref, u_ref, st_ref):
    i = pl.program_id(0)
    h = jnp.concatenate([h2_ref[0], h2_ref[1]], axis=-1)
    agg = jnp.concatenate([agg_ref[0], agg_ref[1]], axis=-1)
    z = (1.0 + eps_ref[0, 0]) * h + agg
    dn = (((1,), (1,)), ((), ()))
    z = jnp.maximum(lax.dot_general(z, w0_ref[...], dn,
                                    preferred_element_type=jnp.float32)
                    + b0_ref[...], 0.0)
    z = jnp.maximum(lax.dot_general(z, w1_ref[...], dn,
                                    preferred_element_type=jnp.float32)
                    + b1_ref[...], 0.0)
    z = jnp.maximum(lax.dot_general(z, w2_ref[...], dn,
                                    preferred_element_type=jnp.float32)
                    + b2_ref[...], 0.0)
    u_ref[...] = z
    st = jnp.concatenate([jnp.sum(z, axis=0, keepdims=True),
                          jnp.sum(z * z, axis=0, keepdims=True)], axis=0)

    @pl.when(i == 0)
    def _():
        st_ref[...] = st

    @pl.when(i > 0)
    def _():
        st_ref[...] += st


def _mlp_call(h2, agg2, W0, b0, W1, b1, W2, b2, eps):
    full = lambda shape: pl.BlockSpec(shape, lambda i: (0,) * len(shape))
    return pl.pallas_call(
        _mlp_body,
        grid=(NB,),
        in_specs=[
            pl.BlockSpec((2, BN, DH), lambda i: (0, i, 0)),
            pl.BlockSpec((2, BN, DH), lambda i: (0, i, 0)),
            full((D, D)), full((1, D)),
            full((D, D)), full((1, D)),
            full((D, D)), full((1, D)),
            full((1, 1)),
        ],
        out_specs=[
            pl.BlockSpec((BN, D), lambda i: (i, 0)),
            pl.BlockSpec((2, D), lambda i: (0, 0)),
        ],
        out_shape=[
            jax.ShapeDtypeStruct((N, D), jnp.float32),
            jax.ShapeDtypeStruct((2, D), jnp.float32),
        ],
    )(h2, agg2, W0, b0.reshape(1, D), W1, b1.reshape(1, D),
      W2, b2.reshape(1, D), eps.reshape(1, 1))


def _bn_pool_body(u_ref, st_ref, gamma_ref, beta_ref, gid_ref,
                  h2_ref, gf_ref):
    i = pl.program_id(0)
    inv_n = 1.0 / N
    mean = st_ref[0:1, :] * inv_n
    var = st_ref[1:2, :] * inv_n - mean * mean
    scale = lax.rsqrt(var + 1e-5) * gamma_ref[...]
    h = (u_ref[...] - mean) * scale + beta_ref[...]
    h2_ref[0] = h[:, :DH]
    h2_ref[1] = h[:, DH:]
    oh = (gid_ref[...] == lax.broadcasted_iota(jnp.int32, (1, G), 1)
          ).astype(jnp.float32)
    part = lax.dot_general(oh, h, (((0,), (0,)), ((), ())),
                           preferred_element_type=jnp.float32)

    @pl.when(i == 0)
    def _():
        gf_ref[...] = part

    @pl.when(i > 0)
    def _():
        gf_ref[...] += part


def _bn_pool_call(u, st, gamma, beta, gid2d):
    return pl.pallas_call(
        _bn_pool_body,
        grid=(NB,),
        in_specs=[
            pl.BlockSpec((BN, D), lambda i: (i, 0)),
            pl.BlockSpec((2, D), lambda i: (0, 0)),
            pl.BlockSpec((1, D), lambda i: (0, 0)),
            pl.BlockSpec((1, D), lambda i: (0, 0)),
            pl.BlockSpec((BN, 1), lambda i: (i, 0)),
        ],
        out_specs=[
            pl.BlockSpec((2, BN, DH), lambda i: (0, i, 0)),
            pl.BlockSpec((G, D), lambda i: (0, 0)),
        ],
        out_shape=[
            jax.ShapeDtypeStruct((2, N, DH), jnp.float32),
            jax.ShapeDtypeStruct((G, D), jnp.float32),
        ],
    )(u, st, gamma.reshape(1, D), beta.reshape(1, D), gid2d)


def _final_body(g0_ref, g1_ref, g2_ref, w_ref, b_ref, out_ref):
    dn = (((1,), (1,)), ((), ()))
    acc = lax.dot_general(g0_ref[...], w_ref[:, 0:D], dn,
                          preferred_element_type=jnp.float32)
    acc += lax.dot_general(g1_ref[...], w_ref[:, D:2 * D], dn,
                           preferred_element_type=jnp.float32)
    acc += lax.dot_general(g2_ref[...], w_ref[:, 2 * D:3 * D], dn,
                           preferred_element_type=jnp.float32)
    out_ref[...] = acc + b_ref[...]


def _final_call(g0, g1, g2, lin_W, lin_b):
    return pl.pallas_call(
        _final_body,
        out_shape=jax.ShapeDtypeStruct((G, NB_CLASSES), jnp.float32),
    )(g0, g1, g2, lin_W, lin_b.reshape(1, NB_CLASSES))


@jax.jit
def kernel(pkt_length, edge_index, node_graph_id, emb_table, W0, b0, W1, b1,
           W2, b2, bn_gamma, bn_beta, eps_gin, lin_W, lin_b):
    idx = (pkt_length + MTU).astype(jnp.int32)
    idx_pad = jnp.zeros((N_CHUNKS * CHUNK,), jnp.int32).at[:N].set(idx)
    idx2d = idx_pad.reshape(N_CHUNKS, CHUNK)
    npad = E_PAD - E
    src_pad = jnp.zeros((npad,), jnp.int32)
    dst_pad = N + (jnp.arange(npad, dtype=jnp.int32) % (AGG_ROWS - N))
    src2d = jnp.concatenate([edge_index[0].astype(jnp.int32), src_pad]
                            ).reshape(E_CHUNKS, CHUNK)
    dst2d = jnp.concatenate([edge_index[1].astype(jnp.int32), dst_pad]
                            ).reshape(E_CHUNKS, CHUNK)
    emb2 = emb_table.reshape(VOCAB, 2, DH).transpose(1, 0, 2)
    gid2d = node_graph_id.astype(jnp.int32).reshape(N, 1)
    zeros = jnp.zeros((ROWS_PER_TILE, DH), jnp.float32)

    h2 = _embed_call(emb2, idx2d)
    h2b = None
    gfs = []
    for it in range(ITERS):
        agg2 = _edge_call_f32(h2, src2d, dst2d, zeros)
        u, st = _mlp_call(h2, agg2, W0, b0, W1, b1, W2, b2, eps_gin)
        h2, gf = _bn_pool_call(u, st, bn_gamma, bn_beta, gid2d)
        gfs.append(gf)
    return _final_call(gfs[0], gfs[1], gfs[2], lin_W, lin_b)


# speedup vs baseline: 1.6357x; 1.0235x over previous
import jax
import jax.numpy as jnp
from jax import lax
from jax.experimental import pallas as pl
from jax.experimental.pallas import tpu as pltpu
from jax.experimental.pallas import tpu_sc as plsc

N = 50000
E = 800000
D = 64
DH = 32
G = 256
VOCAB = 3100
MTU = 1500
NB_CLASSES = 53
ITERS = 3

CHUNK = 128
N_CHUNKS = (N + CHUNK - 1) // CHUNK
N_TAIL = N - (N_CHUNKS - 1) * CHUNK
NSUB = 16

BLK = 8
E_CHUNKS = 6400
E_PAD = E_CHUNKS * CHUNK
CPT = E_CHUNKS // NSUB
BPT = CPT // BLK
AGG_ROWS = 50016
ROWS_PER_TILE = AGG_ROWS // NSUB
OUT_ROWS_PER_TILE = N // NSUB

BN = 5000
NB = N // BN

_mesh = plsc.VectorSubcoreMesh(core_axis_name="c", subcore_axis_name="s")
_sc_params = pltpu.CompilerParams(use_tc_tiling_on_sc=False)
_sc_params_nolayout = pltpu.CompilerParams(use_tc_tiling_on_sc=False,
                                           needs_layout_passes=False)


def _embed_body(emb2_hbm, idx2d_hbm, h2_out, idx_v, rows_v, sem):
    c = lax.axis_index("c")
    s = lax.axis_index("s")
    n_s = (N_CHUNKS - s + NSUB - 1) // NSUB

    def body(i, _):
        j = s + NSUB * i
        pltpu.sync_copy(idx2d_hbm.at[j], idx_v)
        pltpu.async_copy(emb2_hbm.at[c].at[idx_v], rows_v, sem).wait()

        @pl.when(j < N_CHUNKS - 1)
        def _():
            pltpu.sync_copy(rows_v, h2_out.at[c, pl.ds(j * CHUNK, CHUNK)])

        @pl.when(j == N_CHUNKS - 1)
        def _():
            pltpu.sync_copy(rows_v.at[pl.ds(0, N_TAIL)],
                            h2_out.at[c, pl.ds(j * CHUNK, N_TAIL)])
        return 0

    lax.fori_loop(0, n_s, body, 0)


_embed_call = pl.kernel(
    _embed_body,
    out_type=jax.ShapeDtypeStruct((2, N, DH), jnp.float32),
    mesh=_mesh,
    compiler_params=_sc_params,
    scratch_types=[
        pltpu.VMEM((CHUNK,), jnp.int32),
        pltpu.VMEM((CHUNK, DH), jnp.float32),
        pltpu.SemaphoreType.DMA,
    ],
)


def _make_edge_call(bf16_table, nbuf, gd):
    def pipeline(h2_hbm, src2d_hbm, dst2d_hbm, zeros_hbm, agg_out,
                 agg_sp, sidx2, didx2, rowsg, rowsf, isem, sems):
        c = lax.axis_index("c")
        s = lax.axis_index("s")
        gsem = sems[:nbuf]
        ssem = sems[nbuf:]
        base = s * CPT

        def gsrc(p, j):
            return h2_hbm.at[c].at[sidx2.at[p, j]]

        def sdst(p, k):
            return agg_sp.at[didx2.at[p, k]]

        def convert(ks):
            bfr = rowsg.at[ks]
            ffr = rowsf.at[ks]
            mask = jnp.full((16,), -65536, jnp.int32)

            def crow(r8, _):
                for q in range(8):
                    r = r8 * 8 + q
                    xi = plsc.bitcast(bfr[r], jnp.int32)
                    ffr[r, pl.ds(0, 16)] = plsc.bitcast(
                        lax.shift_left(xi, 16), jnp.float32)
                    ffr[r, pl.ds(16, 16)] = plsc.bitcast(
                        lax.bitwise_and(xi, mask), jnp.float32)
                return 0

            lax.fori_loop(0, CHUNK // 8, crow, 0)

        pltpu.sync_copy(zeros_hbm,
                        agg_sp.at[pl.ds(s * ROWS_PER_TILE, ROWS_PER_TILE)])
        plsc.subcore_barrier()

        pltpu.async_copy(src2d_hbm.at[pl.ds(base, BLK)], sidx2.at[0], isem)
        pltpu.async_copy(dst2d_hbm.at[pl.ds(base, BLK)], didx2.at[0], isem)

        def block(b, _):
            p = lax.rem(b, 3)
            boff = base + b * BLK
            pltpu.make_async_copy(src2d_hbm.at[pl.ds(boff, BLK)],
                                  sidx2.at[p], isem).wait()
            pltpu.make_async_copy(dst2d_hbm.at[pl.ds(boff, BLK)],
                                  didx2.at[p], isem).wait()

            @pl.when(b + 1 < BPT)
            def _():
                pn = lax.rem(b + 1, 3)
                noff = boff + BLK
                pltpu.async_copy(src2d_hbm.at[pl.ds(noff, BLK)],
                                 sidx2.at[pn], isem)
                pltpu.async_copy(dst2d_hbm.at[pl.ds(noff, BLK)],
                                 didx2.at[pn], isem)

            def scatter(p2, k):
                ks = k % nbuf
                pltpu.make_async_copy(gsrc(p2, k), rowsg.at[ks],
                                      gsem[ks]).wait()
                if bf16_table:
                    convert(ks)
                pltpu.async_copy(rowsf.at[ks], sdst(p2, k),
                                 ssem[ks], add=True)

            for j in range(BLK):
                slot = j % nbuf
                if j >= nbuf:
                    pltpu.make_async_copy(rowsf.at[slot],
                                          sdst(p, j - nbuf),
                                          ssem[slot]).wait()
                else:
                    @pl.when(b > 0)
                    def _(slot=slot, j=j, p=p):
                        pltpu.make_async_copy(rowsf.at[slot], sdst(p, j),
                                              ssem[slot]).wait()
                pltpu.async_copy(gsrc(p, j), rowsg.at[slot], gsem[slot])
                if j >= gd:
                    scatter(p, j - gd)
            for k in range(BLK - gd, BLK):
                scatter(p, k)
            return 0

        lax.fori_loop(0, BPT, block, 0)
        lastp = (BPT - 1) % 3
        for k in range(BLK - nbuf, BLK):
            ks = k % nbuf
            pltpu.make_async_copy(rowsf.at[ks], sdst(lastp, k),
                                  ssem[ks]).wait()
        plsc.subcore_barrier()
        pltpu.sync_copy(
            agg_sp.at[pl.ds(s * OUT_ROWS_PER_TILE, OUT_ROWS_PER_TILE)],
            agg_out.at[c, pl.ds(s * OUT_ROWS_PER_TILE, OUT_ROWS_PER_TILE)])

    gdtype = jnp.bfloat16 if bf16_table else jnp.float32
    scratch = [
        pltpu.VMEM_SHARED((AGG_ROWS, DH), jnp.float32),
        pltpu.VMEM((3, BLK, CHUNK), jnp.int32),
        pltpu.VMEM((3, BLK, CHUNK), jnp.int32),
        pltpu.VMEM((nbuf, CHUNK, DH), gdtype),
    ]
    if bf16_table:
        scratch.append(pltpu.VMEM((nbuf, CHUNK, DH), jnp.float32))
    scratch += [pltpu.SemaphoreType.DMA] * (1 + 2 * nbuf)

    if bf16_table:
        def body(h2_hbm, src2d_hbm, dst2d_hbm, zeros_hbm, agg_out,
                 agg_sp, sidx2, didx2, rowsg, rowsf, isem, *sems):
            pipeline(h2_hbm, src2d_hbm, dst2d_hbm, zeros_hbm, agg_out,
                     agg_sp, sidx2, didx2, rowsg, rowsf, isem, sems)
    else:
        def body(h2_hbm, src2d_hbm, dst2d_hbm, zeros_hbm, agg_out,
                 agg_sp, sidx2, didx2, rowsg, isem, *sems):
            pipeline(h2_hbm, src2d_hbm, dst2d_hbm, zeros_hbm, agg_out,
                     agg_sp, sidx2, didx2, rowsg, rowsg, isem, sems)

    return pl.kernel(
        body,
        out_type=jax.ShapeDtypeStruct((2, N, DH), jnp.float32),
        mesh=_mesh,
        compiler_params=_sc_params_nolayout if bf16_table else _sc_params,
        scratch_types=scratch,
    )


_edge_call_f32 = _make_edge_call(False, 6, 3)
_edge_call_bf16 = _make_edge_call(True, 4, 2)


def _mlp_body(h2_ref, agg_ref, w0_ref, b0_ref, w1_ref, b1_ref, w2_ref, b2_ref,
              eps_ref, u_ref, st_ref):
    i = pl.program_id(0)
    h = jnp.concatenate([h2_ref[0], h2_ref[1]], axis=-1)
    agg = jnp.concatenate([agg_ref[0], agg_ref[1]], axis=-1)
    z = (1.0 + eps_ref[0, 0]) * h + agg
    dn = (((1,), (1,)), ((), ()))
    z = jnp.maximum(lax.dot_general(z, w0_ref[...], dn,
                                    preferred_element_type=jnp.float32)
                    + b0_ref[...], 0.0)
    z = jnp.maximum(lax.dot_general(z, w1_ref[...], dn,
                                    preferred_element_type=jnp.float32)
                    + b1_ref[...], 0.0)
    z = jnp.maximum(lax.dot_general(z, w2_ref[...], dn,
                                    preferred_element_type=jnp.float32)
                    + b2_ref[...], 0.0)
    u_ref[...] = z
    st = jnp.concatenate([jnp.sum(z, axis=0, keepdims=True),
                          jnp.sum(z * z, axis=0, keepdims=True)], axis=0)

    @pl.when(i == 0)
    def _():
        st_ref[...] = st

    @pl.when(i > 0)
    def _():
        st_ref[...] += st


def _mlp_call(h2, agg2, W0, b0, W1, b1, W2, b2, eps):
    full = lambda shape: pl.BlockSpec(shape, lambda i: (0,) * len(shape))
    return pl.pallas_call(
        _mlp_body,
        grid=(NB,),
        in_specs=[
            pl.BlockSpec((2, BN, DH), lambda i: (0, i, 0)),
            pl.BlockSpec((2, BN, DH), lambda i: (0, i, 0)),
            full((D, D)), full((1, D)),
            full((D, D)), full((1, D)),
            full((D, D)), full((1, D)),
            full((1, 1)),
        ],
        out_specs=[
            pl.BlockSpec((BN, D), lambda i: (i, 0)),
            pl.BlockSpec((2, D), lambda i: (0, 0)),
        ],
        out_shape=[
            jax.ShapeDtypeStruct((N, D), jnp.float32),
            jax.ShapeDtypeStruct((2, D), jnp.float32),
        ],
    )(h2, agg2, W0, b0.reshape(1, D), W1, b1.reshape(1, D),
      W2, b2.reshape(1, D), eps.reshape(1, 1))


def _bn_pool_body(u_ref, st_ref, gamma_ref, beta_ref, gid_ref,
                  h2_ref, gf_ref):
    i = pl.program_id(0)
    inv_n = 1.0 / N
    mean = st_ref[0:1, :] * inv_n
    var = st_ref[1:2, :] * inv_n - mean * mean
    scale = lax.rsqrt(var + 1e-5) * gamma_ref[...]
    h = (u_ref[...] - mean) * scale + beta_ref[...]
    h2_ref[0] = h[:, :DH]
    h2_ref[1] = h[:, DH:]
    oh = (gid_ref[...] == lax.broadcasted_iota(jnp.int32, (1, G), 1)
          ).astype(jnp.float32)
    part = lax.dot_general(oh, h, (((0,), (0,)), ((), ())),
                           preferred_element_type=jnp.float32)

    @pl.when(i == 0)
    def _():
        gf_ref[...] = part

    @pl.when(i > 0)
    def _():
        gf_ref[...] += part


def _bn_pool_call(u, st, gamma, beta, gid2d):
    return pl.pallas_call(
        _bn_pool_body,
        grid=(NB,),
        in_specs=[
            pl.BlockSpec((BN, D), lambda i: (i, 0)),
            pl.BlockSpec((2, D), lambda i: (0, 0)),
            pl.BlockSpec((1, D), lambda i: (0, 0)),
            pl.BlockSpec((1, D), lambda i: (0, 0)),
            pl.BlockSpec((BN, 1), lambda i: (i, 0)),
        ],
        out_specs=[
            pl.BlockSpec((2, BN, DH), lambda i: (0, i, 0)),
            pl.BlockSpec((G, D), lambda i: (0, 0)),
        ],
        out_shape=[
            jax.ShapeDtypeStruct((2, N, DH), jnp.float32),
            jax.ShapeDtypeStruct((G, D), jnp.float32),
        ],
    )(u, st, gamma.reshape(1, D), beta.reshape(1, D), gid2d)


def _final_body(g0_ref, g1_ref, g2_ref, w_ref, b_ref, out_ref):
    dn = (((1,), (1,)), ((), ()))
    acc = lax.dot_general(g0_ref[...], w_ref[:, 0:D], dn,
                          preferred_element_type=jnp.float32)
    acc += lax.dot_general(g1_ref[...], w_ref[:, D:2 * D], dn,
                           preferred_element_type=jnp.float32)
    acc += lax.dot_general(g2_ref[...], w_ref[:, 2 * D:3 * D], dn,
                           preferred_element_type=jnp.float32)
    out_ref[...] = acc + b_ref[...]


def _final_call(g0, g1, g2, lin_W, lin_b):
    return pl.pallas_call(
        _final_body,
        out_shape=jax.ShapeDtypeStruct((G, NB_CLASSES), jnp.float32),
    )(g0, g1, g2, lin_W, lin_b.reshape(1, NB_CLASSES))


@jax.jit
def kernel(pkt_length, edge_index, node_graph_id, emb_table, W0, b0, W1, b1,
           W2, b2, bn_gamma, bn_beta, eps_gin, lin_W, lin_b):
    idx = (pkt_length + MTU).astype(jnp.int32)
    idx_pad = jnp.zeros((N_CHUNKS * CHUNK,), jnp.int32).at[:N].set(idx)
    idx2d = idx_pad.reshape(N_CHUNKS, CHUNK)
    npad = E_PAD - E
    src_pad = jnp.zeros((npad,), jnp.int32)
    dst_pad = N + (jnp.arange(npad, dtype=jnp.int32) % (AGG_ROWS - N))
    src2d = jnp.concatenate([edge_index[0].astype(jnp.int32), src_pad]
                            ).reshape(E_CHUNKS, CHUNK)
    dst2d = jnp.concatenate([edge_index[1].astype(jnp.int32), dst_pad]
                            ).reshape(E_CHUNKS, CHUNK)
    emb2 = emb_table.reshape(VOCAB, 2, DH).transpose(1, 0, 2)
    gid2d = node_graph_id.astype(jnp.int32).reshape(N, 1)
    zeros = jnp.zeros((ROWS_PER_TILE, DH), jnp.float32)

    h2 = _embed_call(emb2, idx2d)
    h2b = None
    gfs = []
    for it in range(ITERS):
        agg2 = _edge_call_f32(h2, src2d, dst2d, zeros)
        u, st = _mlp_call(h2, agg2, W0, b0, W1, b1, W2, b2, eps_gin)
        h2, gf = _bn_pool_call(u, st, bn_gamma, bn_beta, gid2d)
        gfs.append(gf)
    return _final_call(gfs[0], gfs[1], gfs[2], lin_W, lin_b)
